# initial kernel scaffold (unmeasured)
import jax
import jax.numpy as jnp
from jax import lax
from jax.experimental import pallas as pl
from jax.experimental.pallas import tpu as pltpu

N_DEV = 32
B = 2
SQ_L = 256
SKV = 256
HQ = 128
H_L = 4
DH = 64
DM = 512
DF_L = H_L * DH
NEG = -1e9


def kernel(x, Wq, K_ext, V_ext, Wo):
    K_t = jnp.transpose(K_ext, (0, 2, 1, 3))
    V_t = jnp.transpose(V_ext, (0, 2, 1, 3))

    def body(x_ref, wq_ref, k_ref, v_ref, wo_ref, out_ref,
             wq_buf, wo_buf, ctx_ref,
             send_q, recv_q, send_o, recv_o):
        my = lax.axis_index("i")
        right = lax.rem(my + 1, N_DEV)
        left = lax.rem(my + N_DEV - 1, N_DEV)

        barrier = pltpu.get_barrier_semaphore()
        pl.semaphore_signal(barrier, inc=1, device_id=(left,),
                            device_id_type=pl.DeviceIdType.MESH)
        pl.semaphore_signal(barrier, inc=1, device_id=(right,),
                            device_id_type=pl.DeviceIdType.MESH)
        pl.semaphore_wait(barrier, 2)

        x2 = x_ref[...].reshape(B * SQ_L, DM)
        qb = lax.broadcasted_iota(jnp.int32, (SQ_L, SKV), 0) // 64
        kb = lax.broadcasted_iota(jnp.int32, (SQ_L, SKV), 1) // 64
        mask = qb == kb

        def stage(j, wq, wo, first):
            q = jnp.dot(x2, wq, preferred_element_type=jnp.float32)
            for b in range(B):
                kj = k_ref[b, pl.ds(j * H_L, H_L)]
                vj = v_ref[b, pl.ds(j * H_L, H_L)]
                for h in range(H_L):
                    qbh = q[b * SQ_L:(b + 1) * SQ_L, h * DH:(h + 1) * DH]
                    s = lax.dot_general(
                        qbh, kj[h],
                        (((1,), (1,)), ((), ())),
                        preferred_element_type=jnp.float32,
                    ) * 0.125
                    s = jnp.where(mask, s, NEG)
                    m = jnp.max(s, axis=1, keepdims=True)
                    w = jnp.exp(s - m)
                    w = w / jnp.sum(w, axis=1, keepdims=True)
                    c = jnp.dot(w, vj[h], preferred_element_type=jnp.float32)
                    ctx_ref[b * SQ_L:(b + 1) * SQ_L, h * DH:(h + 1) * DH] = c
            partial = jnp.dot(ctx_ref[...], wo,
                              preferred_element_type=jnp.float32)
            partial = partial.reshape(B, SQ_L, DM)
            if first:
                out_ref[...] = partial
            else:
                out_ref[...] = out_ref[...] + partial

        wq_buf[0] = wq_ref[...]
        wo_buf[0] = wo_ref[...]
        stage(my, wq_ref[...], wo_ref[...], first=True)

        def hop(h, carry):
            send_slot = lax.rem(h, 2)
            recv_slot = lax.rem(h + 1, 2)
            rq = pltpu.make_async_remote_copy(
                src_ref=wq_buf.at[send_slot],
                dst_ref=wq_buf.at[recv_slot],
                send_sem=send_q.at[send_slot],
                recv_sem=recv_q.at[recv_slot],
                device_id=(right,),
                device_id_type=pl.DeviceIdType.MESH,
            )
            ro = pltpu.make_async_remote_copy(
                src_ref=wo_buf.at[send_slot],
                dst_ref=wo_buf.at[recv_slot],
                send_sem=send_o.at[send_slot],
                recv_sem=recv_o.at[recv_slot],
                device_id=(right,),
                device_id_type=pl.DeviceIdType.MESH,
            )
            rq.start()
            ro.start()
            rq.wait()
            ro.wait()
            j = lax.rem(my - h - 1 + 2 * N_DEV, N_DEV)
            stage(j, wq_buf[recv_slot], wo_buf[recv_slot], first=False)
            return carry

        lax.fori_loop(0, N_DEV - 1, hop, 0)

    return pl.pallas_call(
        body,
        out_shape=jax.ShapeDtypeStruct((B, SQ_L, DM), jnp.float32),
        in_specs=[pl.BlockSpec(memory_space=pltpu.VMEM)] * 5,
        out_specs=pl.BlockSpec(memory_space=pltpu.VMEM),
        scratch_shapes=[
            pltpu.VMEM((2, DM, DF_L), jnp.float32),
            pltpu.VMEM((2, DF_L, DM), jnp.float32),
            pltpu.VMEM((B * SQ_L, DF_L), jnp.float32),
            pltpu.SemaphoreType.DMA((2,)),
            pltpu.SemaphoreType.DMA((2,)),
            pltpu.SemaphoreType.DMA((2,)),
            pltpu.SemaphoreType.DMA((2,)),
        ],
        compiler_params=pltpu.CompilerParams(collective_id=0),
    )(x, Wq, K_t, V_t, Wo)


# baseline (device time: 578450 ns/iter reference)
import jax
import jax.numpy as jnp
from jax import lax
from jax.experimental import pallas as pl
from jax.experimental.pallas import tpu as pltpu

N_DEV = 32
B = 2
SQ_L = 256
SKV = 256
HQ = 128
H_L = 4
DH = 64
DM = 512
DF_L = H_L * DH
NEG = -1e9


def kernel(x, Wq, K_ext, V_ext, Wo):
    K_t = jnp.transpose(K_ext, (0, 2, 3, 1))
    V_t = jnp.transpose(V_ext, (0, 2, 3, 1))

    def body(x_ref, wq_ref, k_ref, v_ref, wo_ref, out_ref,
             wq_buf, wo_buf, ctx_ref,
             send_q, recv_q, send_o, recv_o):
        my = lax.axis_index("i")
        right = lax.rem(my + 1, N_DEV)
        left = lax.rem(my + N_DEV - 1, N_DEV)

        barrier = pltpu.get_barrier_semaphore()
        pl.semaphore_signal(barrier, inc=1, device_id=(left,),
                            device_id_type=pl.DeviceIdType.MESH)
        pl.semaphore_signal(barrier, inc=1, device_id=(right,),
                            device_id_type=pl.DeviceIdType.MESH)
        pl.semaphore_wait(barrier, 2)

        x2 = x_ref[...].reshape(B * SQ_L, DM)
        qb = lax.broadcasted_iota(jnp.int32, (SQ_L, SKV), 0) // 64
        kb = lax.broadcasted_iota(jnp.int32, (SQ_L, SKV), 1) // 64
        mask = qb == kb

        def stage(j, wq, wo, first):
            q = jnp.dot(x2, wq, preferred_element_type=jnp.float32)
            for b in range(B):
                kj = k_ref[b, pl.ds(j * H_L, H_L)]
                vj = v_ref[b, pl.ds(j * H_L, H_L)]
                for h in range(H_L):
                    qbh = q[b * SQ_L:(b + 1) * SQ_L, h * DH:(h + 1) * DH]
                    s = jnp.dot(qbh, kj[h],
                                preferred_element_type=jnp.float32) * 0.125
                    s = jnp.where(mask, s, NEG)
                    m = jnp.max(s, axis=1, keepdims=True)
                    w = jnp.exp(s - m)
                    w = w / jnp.sum(w, axis=1, keepdims=True)
                    ct = lax.dot_general(
                        vj[h], w,
                        (((1,), (1,)), ((), ())),
                        preferred_element_type=jnp.float32,
                    )
                    ctx_ref[h * DH:(h + 1) * DH,
                            b * SQ_L:(b + 1) * SQ_L] = ct
            partial = lax.dot_general(
                ctx_ref[...], wo,
                (((0,), (0,)), ((), ())),
                preferred_element_type=jnp.float32,
            )
            partial = partial.reshape(B, SQ_L, DM)
            if first:
                out_ref[...] = partial
            else:
                out_ref[...] = out_ref[...] + partial

        wq_buf[0] = wq_ref[...]
        wo_buf[0] = wo_ref[...]
        stage(my, wq_ref[...], wo_ref[...], first=True)

        def hop(h, carry):
            send_slot = lax.rem(h, 2)
            recv_slot = lax.rem(h + 1, 2)
            rq = pltpu.make_async_remote_copy(
                src_ref=wq_buf.at[send_slot],
                dst_ref=wq_buf.at[recv_slot],
                send_sem=send_q.at[send_slot],
                recv_sem=recv_q.at[recv_slot],
                device_id=(right,),
                device_id_type=pl.DeviceIdType.MESH,
            )
            ro = pltpu.make_async_remote_copy(
                src_ref=wo_buf.at[send_slot],
                dst_ref=wo_buf.at[recv_slot],
                send_sem=send_o.at[send_slot],
                recv_sem=recv_o.at[recv_slot],
                device_id=(right,),
                device_id_type=pl.DeviceIdType.MESH,
            )
            rq.start()
            ro.start()
            rq.wait()
            ro.wait()
            j = lax.rem(my - h - 1 + 2 * N_DEV, N_DEV)
            stage(j, wq_buf[recv_slot], wo_buf[recv_slot], first=False)
            return carry

        lax.fori_loop(0, N_DEV - 1, hop, 0)

    return pl.pallas_call(
        body,
        out_shape=jax.ShapeDtypeStruct((B, SQ_L, DM), jnp.float32),
        in_specs=[pl.BlockSpec(memory_space=pltpu.VMEM)] * 5,
        out_specs=pl.BlockSpec(memory_space=pltpu.VMEM),
        scratch_shapes=[
            pltpu.VMEM((2, DM, DF_L), jnp.float32),
            pltpu.VMEM((2, DF_L, DM), jnp.float32),
            pltpu.VMEM((DF_L, B * SQ_L), jnp.float32),
            pltpu.SemaphoreType.DMA((2,)),
            pltpu.SemaphoreType.DMA((2,)),
            pltpu.SemaphoreType.DMA((2,)),
            pltpu.SemaphoreType.DMA((2,)),
        ],
        compiler_params=pltpu.CompilerParams(collective_id=0),
    )(x, Wq, K_t, V_t, Wo)


# device time: 224801 ns/iter; 2.5732x vs baseline; 2.5732x over previous
import jax
import jax.numpy as jnp
from jax import lax
from jax.experimental import pallas as pl
from jax.experimental.pallas import tpu as pltpu

N_DEV = 32
B = 2
SQ_L = 256
SKV = 256
HQ = 128
H_L = 4
DH = 64
DM = 512
DF_L = H_L * DH
NEG = -1e9
R_HOPS = 16
L_HOPS = 15


def kernel(x, Wq, K_ext, V_ext, Wo):
    K_t = jnp.transpose(K_ext, (0, 2, 3, 1)).astype(jnp.bfloat16)
    V_t = jnp.transpose(V_ext, (0, 2, 3, 1)).astype(jnp.bfloat16)

    def body(x_ref, wq_ref, k_ref, v_ref, wo_ref, out_ref,
             rq_buf, ro_buf, lq_buf, lo_buf, ctx_ref,
             rq_send, rq_recv, ro_send, ro_recv,
             lq_send, lq_recv, lo_send, lo_recv):
        my = lax.axis_index("i")
        right = lax.rem(my + 1, N_DEV)
        left = lax.rem(my + N_DEV - 1, N_DEV)

        barrier = pltpu.get_barrier_semaphore()
        pl.semaphore_signal(barrier, inc=1, device_id=(left,),
                            device_id_type=pl.DeviceIdType.MESH)
        pl.semaphore_signal(barrier, inc=1, device_id=(right,),
                            device_id_type=pl.DeviceIdType.MESH)
        pl.semaphore_wait(barrier, 2)

        x2 = x_ref[...].reshape(B * SQ_L, DM).astype(jnp.bfloat16)
        qb = lax.broadcasted_iota(jnp.int32, (SQ_L, SKV), 0) // 64
        kb = lax.broadcasted_iota(jnp.int32, (SQ_L, SKV), 1) // 64
        mask = qb == kb

        def desc(buf, send_sems, recv_sems, sslot, rslot, dev):
            return pltpu.make_async_remote_copy(
                src_ref=buf.at[sslot],
                dst_ref=buf.at[rslot],
                send_sem=send_sems.at[sslot],
                recv_sem=recv_sems.at[rslot],
                device_id=(dev,),
                device_id_type=pl.DeviceIdType.MESH,
            )

        def r_descs(sslot, rslot):
            return (desc(rq_buf, rq_send, rq_recv, sslot, rslot, right),
                    desc(ro_buf, ro_send, ro_recv, sslot, rslot, right))

        def l_descs(sslot, rslot):
            return (desc(lq_buf, lq_send, lq_recv, sslot, rslot, left),
                    desc(lo_buf, lo_send, lo_recv, sslot, rslot, left))

        def stage(j, wq, wo, first):
            q = jnp.dot(x2, wq, preferred_element_type=jnp.float32)
            q = q.astype(jnp.bfloat16)
            for b in range(B):
                kj = k_ref[b, pl.ds(j * H_L, H_L)]
                vj = v_ref[b, pl.ds(j * H_L, H_L)]
                for h in range(H_L):
                    qbh = q[b * SQ_L:(b + 1) * SQ_L, h * DH:(h + 1) * DH]
                    s = jnp.dot(qbh, kj[h],
                                preferred_element_type=jnp.float32) * 0.125
                    s = jnp.where(mask, s, NEG)
                    m = jnp.max(s, axis=1, keepdims=True)
                    w = jnp.exp(s - m)
                    w = (w / jnp.sum(w, axis=1, keepdims=True)
                         ).astype(jnp.bfloat16)
                    ct = lax.dot_general(
                        vj[h], w,
                        (((1,), (1,)), ((), ())),
                        preferred_element_type=jnp.float32,
                    )
                    ctx_ref[h * DH:(h + 1) * DH,
                            b * SQ_L:(b + 1) * SQ_L] = ct
            partial = lax.dot_general(
                ctx_ref[...].astype(jnp.bfloat16), wo,
                (((0,), (0,)), ((), ())),
                preferred_element_type=jnp.float32,
            )
            partial = partial.reshape(B, SQ_L, DM)
            if first:
                out_ref[...] = partial
            else:
                out_ref[...] = out_ref[...] + partial

        wq_bf = wq_ref[...].astype(jnp.bfloat16)
        wo_bf = wo_ref[...].astype(jnp.bfloat16)
        rq_buf[0] = wq_bf
        ro_buf[0] = wo_bf
        lq_buf[0] = wq_bf
        lo_buf[0] = wo_bf

        for d in r_descs(0, 1) + l_descs(0, 1):
            d.start()
        stage(my, wq_bf, wo_bf, first=True)

        def hop_ops(h, r_send=True, l_recv=True, l_send=True,
                    wait_old_send=True):
            h = jnp.int32(h)
            rr = lax.rem(h + 1, 3)
            rn = lax.rem(h + 2, 3)

            for d in r_descs(rn, rr):
                d.wait_recv()
            if r_send:
                if wait_old_send:
                    for d in r_descs(rr, rn):
                        d.wait_send()
                for d in r_descs(rr, rn):
                    d.start()

            if l_recv:
                for d in l_descs(rn, rr):
                    d.wait_recv()
                if l_send:
                    if wait_old_send:
                        for d in l_descs(rr, rn):
                            d.wait_send()
                    for d in l_descs(rr, rn):
                        d.start()

            j_r = lax.rem(my - h - 1 + 2 * N_DEV, N_DEV)
            stage(j_r, rq_buf[rr], ro_buf[rr], first=False)
            if l_recv:
                j_l = lax.rem(my + h + 1, N_DEV)
                stage(j_l, lq_buf[rr], lo_buf[rr], first=False)

        hop_ops(0, wait_old_send=False)
        hop_ops(1, wait_old_send=False)

        def hop(h, carry):
            hop_ops(h)
            return carry

        lax.fori_loop(2, 13, hop, 0)
        hop_ops(13)
        hop_ops(14, l_send=False)
        hop_ops(15, r_send=False, l_recv=False)

        for slot in (0, 1, 2):
            for d in r_descs(slot, 0) + l_descs(slot, 0):
                d.wait_send()

    comm = jnp.bfloat16
    return pl.pallas_call(
        body,
        out_shape=jax.ShapeDtypeStruct((B, SQ_L, DM), jnp.float32),
        in_specs=[pl.BlockSpec(memory_space=pltpu.VMEM)] * 5,
        out_specs=pl.BlockSpec(memory_space=pltpu.VMEM),
        scratch_shapes=[
            pltpu.VMEM((3, DM, DF_L), comm),
            pltpu.VMEM((3, DF_L, DM), comm),
            pltpu.VMEM((3, DM, DF_L), comm),
            pltpu.VMEM((3, DF_L, DM), comm),
            pltpu.VMEM((DF_L, B * SQ_L), jnp.float32),
            pltpu.SemaphoreType.DMA((3,)),
            pltpu.SemaphoreType.DMA((3,)),
            pltpu.SemaphoreType.DMA((3,)),
            pltpu.SemaphoreType.DMA((3,)),
            pltpu.SemaphoreType.DMA((3,)),
            pltpu.SemaphoreType.DMA((3,)),
            pltpu.SemaphoreType.DMA((3,)),
            pltpu.SemaphoreType.DMA((3,)),
        ],
        compiler_params=pltpu.CompilerParams(collective_id=0),
    )(x, Wq, K_t, V_t, Wo)
